# Initial kernel scaffold; baseline (speedup 1.0000x reference)
import functools

import jax
import jax.numpy as jnp
from jax.experimental import pallas as pl
from jax.experimental.pallas import tpu as pltpu

_B, _NQ, _NK, _DIM = 8, 32, 8192, 768
_ID = 64
_BK = 1024
_NKB = _NK // _BK
_SCALE = _ID ** (-0.5)
_NEG = jnp.float32(-3.0e38)


def _body(q_ref, wq_ref, k_ref, wk_ref, out_ref, qh_s, s_s):
    j = pl.program_id(1)

    @pl.when(j == 0)
    def _():
        qh_s[...] = jax.lax.dot_general(
            q_ref[0], wq_ref[...], (((1,), (0,)), ((), ())),
            preferred_element_type=jnp.float32) * _SCALE

    kp = jax.lax.dot_general(
        k_ref[0], wk_ref[...], (((1,), (0,)), ((), ())),
        preferred_element_type=jnp.float32)
    s = jax.lax.dot_general(
        qh_s[...], kp, (((1,), (1,)), ((), ())),
        preferred_element_type=jnp.float32)
    s_s[:, pl.ds(j * _BK, _BK)] = s

    @pl.when(j == _NKB - 1)
    def _():
        S = s_s[...]
        m1 = jnp.max(S, axis=-1, keepdims=True)
        S2 = jnp.where(S < m1, S, _NEG)
        m2 = jnp.max(S2, axis=-1, keepdims=True)
        S3 = jnp.where(S2 < m2, S2, _NEG)
        m3 = jnp.max(S3, axis=-1, keepdims=True)
        S4 = jnp.where(S3 < m3, S3, _NEG)
        m4 = jnp.max(S4, axis=-1, keepdims=True)
        one = jnp.float32(1.0)
        zero = jnp.float32(0.0)
        c1 = jnp.sum(jnp.where(S == m1, one, zero), axis=-1, keepdims=True)
        c2 = jnp.sum(jnp.where(S == m2, one, zero), axis=-1, keepdims=True)
        c3 = jnp.sum(jnp.where(S == m3, one, zero), axis=-1, keepdims=True)
        thr = jnp.where(c1 >= 4, m1,
              jnp.where(c1 + c2 >= 4, m2,
              jnp.where(c1 + c2 + c3 >= 4, m3, m4)))
        P = jnp.where(S >= thr, jnp.exp(S - m1), zero)
        denom = jnp.sum(P, axis=-1, keepdims=True)
        out_ref[0, 0] = P * (one / denom)


@functools.partial(jax.jit, static_argnames=("interpret",))
def _run(q, k, Wq, Wk, interpret=False):
    return pl.pallas_call(
        _body,
        grid=(_B, _NKB),
        in_specs=[
            pl.BlockSpec((1, _NQ, _DIM), lambda b, j: (b, 0, 0)),
            pl.BlockSpec((_DIM, _ID), lambda b, j: (0, 0)),
            pl.BlockSpec((1, _BK, _DIM), lambda b, j: (b, j, 0)),
            pl.BlockSpec((_DIM, _ID), lambda b, j: (0, 0)),
        ],
        out_specs=pl.BlockSpec((1, 1, _NQ, _NK), lambda b, j: (b, 0, 0, 0)),
        out_shape=jax.ShapeDtypeStruct((_B, 1, _NQ, _NK), jnp.float32),
        scratch_shapes=[
            pltpu.VMEM((_NQ, _ID), jnp.float32),
            pltpu.VMEM((_NQ, _NK), jnp.float32),
        ],
        interpret=interpret,
    )(q, Wq, k, Wk)


def kernel(q, k, v, Wq, Wk):
    del v
    return _run(q, k, Wq, Wk)


# speedup vs baseline: 11.4635x; 11.4635x over previous
import functools

import jax
import jax.numpy as jnp
from jax.experimental import pallas as pl
from jax.experimental.pallas import tpu as pltpu

_B, _NQ, _NK, _DIM = 8, 32, 8192, 768
_ID = 64
_BK = 1024
_NKB = _NK // _BK
_SCALE = _ID ** (-0.5)
_NEG = -3.0e38


def _body(q_ref, wq_ref, k_ref, wk_ref, out_ref, qh_s, s_s):
    j = pl.program_id(1)

    @pl.when(j == 0)
    def _():
        qh_s[...] = jax.lax.dot_general(
            q_ref[0], wq_ref[...], (((1,), (0,)), ((), ())),
            preferred_element_type=jnp.float32) * _SCALE

    kp = jax.lax.dot_general(
        k_ref[0], wk_ref[...], (((1,), (0,)), ((), ())),
        preferred_element_type=jnp.float32)
    s = jax.lax.dot_general(
        qh_s[...], kp, (((1,), (1,)), ((), ())),
        preferred_element_type=jnp.float32)
    s_s[:, pl.ds(j * _BK, _BK)] = s

    @pl.when(j == _NKB - 1)
    def _():
        S = s_s[...]
        m1 = jnp.max(S, axis=-1, keepdims=True)
        S2 = jnp.where(S < m1, S, _NEG)
        m2 = jnp.max(S2, axis=-1, keepdims=True)
        S3 = jnp.where(S2 < m2, S2, _NEG)
        m3 = jnp.max(S3, axis=-1, keepdims=True)
        S4 = jnp.where(S3 < m3, S3, _NEG)
        m4 = jnp.max(S4, axis=-1, keepdims=True)
        c1 = jnp.sum(jnp.where(S == m1, 1.0, 0.0), axis=-1, keepdims=True)
        c2 = jnp.sum(jnp.where(S == m2, 1.0, 0.0), axis=-1, keepdims=True)
        c3 = jnp.sum(jnp.where(S == m3, 1.0, 0.0), axis=-1, keepdims=True)
        thr = jnp.where(c1 >= 4.0, m1,
              jnp.where(c1 + c2 >= 4.0, m2,
              jnp.where(c1 + c2 + c3 >= 4.0, m3, m4)))
        P = jnp.where(S >= thr, jnp.exp(S - m1), 0.0)
        denom = jnp.sum(P, axis=-1, keepdims=True)
        out_ref[0, 0] = P * (1.0 / denom)


@functools.partial(jax.jit, static_argnames=("interpret",))
def _run(q, k, Wq, Wk, interpret=False):
    return pl.pallas_call(
        _body,
        grid=(_B, _NKB),
        in_specs=[
            pl.BlockSpec((1, _NQ, _DIM), lambda b, j: (b, 0, 0)),
            pl.BlockSpec((_DIM, _ID), lambda b, j: (0, 0)),
            pl.BlockSpec((1, _BK, _DIM), lambda b, j: (b, j, 0)),
            pl.BlockSpec((_DIM, _ID), lambda b, j: (0, 0)),
        ],
        out_specs=pl.BlockSpec((1, 1, _NQ, _NK), lambda b, j: (b, 0, 0, 0)),
        out_shape=jax.ShapeDtypeStruct((_B, 1, _NQ, _NK), jnp.float32),
        scratch_shapes=[
            pltpu.VMEM((_NQ, _ID), jnp.float32),
            pltpu.VMEM((_NQ, _NK), jnp.float32),
        ],
        interpret=interpret,
    )(q, Wq, k, Wk)


def kernel(q, k, v, Wq, Wk):
    del v
    return _run(q, k, Wq, Wk)
